# baseline (speedup 1.0000x reference)
import functools

import jax
import jax.numpy as jnp
from jax import lax
from jax.experimental import pallas as pl
from jax.experimental.pallas import tpu as pltpu
from jax.experimental.pallas import tpu_sc as plsc

BINS = 10
EPS = 1e-12
DELTA = 1.0 / BINS
N = 4096
NC = 2
NS = 16
L = 16
NW = NC * NS
BI = 512

_mesh = plsc.VectorSubcoreMesh(
    core_axis_name="c", subcore_axis_name="s", num_cores=NC, num_subcores=NS)


@functools.partial(
    pl.kernel,
    out_type=jax.ShapeDtypeStruct((NW, L), jnp.float32),
    mesh=_mesh,
    scratch_types=[pltpu.VMEM((L,), jnp.float32)],
)
def _sc_kernel(x_hbm, z_hbm, out_hbm, p_v):
    wid = lax.axis_index("s") * NC + lax.axis_index("c")
    p_v[...] = jnp.zeros((L,), jnp.float32)
    pltpu.sync_copy(p_v, out_hbm.at[wid])


def _tc_body(xc_ref, zc_ref, xr_ref, zr_ref, out_ref):
    xc = xc_ref[...]
    zc = zc_ref[...]
    xr = xr_ref[...]
    zr = zr_ref[...]
    gc = jnp.abs(jax.nn.sigmoid(xc) - zc)
    gr = jnp.abs(jax.nn.sigmoid(xr) - zr)
    close = (jnp.abs(gc - gr) <= DELTA).astype(jnp.float32)
    cnt = jnp.sum(close, axis=1, keepdims=True)
    gd = cnt / DELTA
    beta = N / (gd + EPS)
    loss = jnp.maximum(xc, 0.0) - xc * zc + jnp.log1p(jnp.exp(-jnp.abs(xc)))
    out_ref[...] = jnp.sum(beta * loss).reshape(1, 1, 1)


def kernel(logits, targets):
    sc_partials = _sc_kernel(logits, targets)
    xc = logits.reshape(N, 1)
    zc = targets.reshape(N, 1)
    xr = logits.reshape(1, N)
    zr = targets.reshape(1, N)
    tc_partials = pl.pallas_call(
        _tc_body,
        grid=(N // BI,),
        in_specs=[
            pl.BlockSpec((BI, 1), lambda i: (i, 0)),
            pl.BlockSpec((BI, 1), lambda i: (i, 0)),
            pl.BlockSpec((1, N), lambda i: (0, 0)),
            pl.BlockSpec((1, N), lambda i: (0, 0)),
        ],
        out_specs=pl.BlockSpec((1, 1, 1), lambda i: (i, 0, 0)),
        out_shape=jax.ShapeDtypeStruct((N // BI, 1, 1), jnp.float32),
    )(xc, zc, xr, zr)
    return (jnp.sum(sc_partials) + jnp.sum(tc_partials)) / N


# speedup vs baseline: 1.0804x; 1.0804x over previous
import functools

import jax
import jax.numpy as jnp
from jax import lax
from jax.experimental import pallas as pl
from jax.experimental.pallas import tpu as pltpu
from jax.experimental.pallas import tpu_sc as plsc

BINS = 10
EPS = 1e-12
DELTA = 1.0 / BINS
N = 4096
NC = 2
NS = 16
L = 16
NW = NC * NS

NSC = 1024
NTC = N - NSC
CHUNK = NSC // NW
IV = CHUNK // L
UR = 4
BI = 512


def _log1p_exp_neg(ax):
    t = jnp.exp(-ax)
    u = t / (2.0 + t)
    u2 = u * u
    s = jnp.float32(1.0 / 13.0)
    for c in (1.0 / 11.0, 1.0 / 9.0, 1.0 / 7.0, 1.0 / 5.0, 1.0 / 3.0, 1.0):
        s = s * u2 + jnp.float32(c)
    return 2.0 * u * s


_mesh = plsc.VectorSubcoreMesh(
    core_axis_name="c", subcore_axis_name="s", num_cores=NC, num_subcores=NS)


@functools.partial(
    pl.kernel,
    out_type=jax.ShapeDtypeStruct((NW, L), jnp.float32),
    mesh=_mesh,
    scratch_types=[
        pltpu.VMEM((N,), jnp.float32),
        pltpu.VMEM((N,), jnp.float32),
        pltpu.VMEM((N + L,), jnp.float32),
        pltpu.VMEM((L,), jnp.float32),
    ],
)
def _sc_kernel(x_hbm, z_hbm, out_hbm, x_v, z_v, g_v, p_v):
    wid = lax.axis_index("s") * NC + lax.axis_index("c")
    pltpu.sync_copy(x_hbm, x_v)
    pltpu.sync_copy(z_hbm, z_v)

    def g_body(k, carry):
        x = x_v[pl.ds(k * L, L)]
        z = z_v[pl.ds(k * L, L)]
        pred = 1.0 / (1.0 + jnp.exp(-x))
        g_v[pl.ds(k * L, L)] = jnp.abs(pred - z)
        return carry

    lax.fori_loop(0, N // L, g_body, 0)
    g_v[pl.ds(N, L)] = g_v[pl.ds(0, L)]

    base = NTC + wid * CHUNK
    gi = [g_v[pl.ds(base + b * L, L)] for b in range(IV)]

    def j_body(t, accs):
        accs = list(accs)
        for r in range(UR):
            gw = g_v[pl.ds(t * UR + r, L)]
            for b in range(IV):
                m = jnp.abs(gw - gi[b]) <= DELTA
                accs[b] = accs[b] + jnp.where(m, 1.0, 0.0)
        return tuple(accs)

    zero = jnp.zeros((L,), jnp.float32)
    accs = lax.fori_loop(0, N // UR, j_body, tuple(zero for _ in range(IV)))

    psum = jnp.zeros((L,), jnp.float32)
    for b in range(IV):
        gd = accs[b] / DELTA
        beta = N / (gd + EPS)
        x = x_v[pl.ds(base + b * L, L)]
        z = z_v[pl.ds(base + b * L, L)]
        loss = jnp.maximum(x, 0.0) - x * z + _log1p_exp_neg(jnp.abs(x))
        psum = psum + beta * loss
    p_v[...] = psum
    pltpu.sync_copy(p_v, out_hbm.at[wid])


def _tc_body(xc_ref, zc_ref, xr_ref, zr_ref, out_ref):
    xc = xc_ref[...]
    zc = zc_ref[...]
    xr = xr_ref[...]
    zr = zr_ref[...]
    gc = jnp.abs(jax.nn.sigmoid(xc) - zc)
    gr = jnp.abs(jax.nn.sigmoid(xr) - zr)
    close = (jnp.abs(gc - gr) <= DELTA).astype(jnp.float32)
    cnt = jnp.sum(close, axis=1, keepdims=True)
    gd = cnt / DELTA
    beta = N / (gd + EPS)
    loss = jnp.maximum(xc, 0.0) - xc * zc + jnp.log1p(jnp.exp(-jnp.abs(xc)))

    @pl.when(pl.program_id(0) == 0)
    def _():
        out_ref[...] = jnp.zeros((1, 1, 1), jnp.float32)

    out_ref[...] += jnp.sum(beta * loss).reshape(1, 1, 1)


def kernel(logits, targets):
    sc_partials = _sc_kernel(logits, targets)

    xc = logits.reshape(N, 1)
    zc = targets.reshape(N, 1)
    xr = logits.reshape(1, N)
    zr = targets.reshape(1, N)
    tc_partials = pl.pallas_call(
        _tc_body,
        grid=(NTC // BI,),
        in_specs=[
            pl.BlockSpec((BI, 1), lambda i: (i, 0)),
            pl.BlockSpec((BI, 1), lambda i: (i, 0)),
            pl.BlockSpec((1, N), lambda i: (0, 0)),
            pl.BlockSpec((1, N), lambda i: (0, 0)),
        ],
        out_specs=pl.BlockSpec((1, 1, 1), lambda i: (0, 0, 0)),
        out_shape=jax.ShapeDtypeStruct((1, 1, 1), jnp.float32),
    )(xc, zc, xr, zr)
    return (jnp.sum(sc_partials) + tc_partials[0, 0, 0]) / N


# speedup vs baseline: 1.1814x; 1.0936x over previous
import functools

import jax
import jax.numpy as jnp
from jax import lax
from jax.experimental import pallas as pl
from jax.experimental.pallas import tpu as pltpu
from jax.experimental.pallas import tpu_sc as plsc

BINS = 10
EPS = 1e-12
DELTA = 1.0 / BINS
N = 4096
NC = 2
NS = 16
L = 16
NW = NC * NS

NSC = 1024
NTC = N - NSC
CHUNK = NSC // NW
IV = CHUNK // L
UR = 4
BI = 512


def _log1p_exp_neg(ax):
    t = jnp.exp(-ax)
    u = t / (2.0 + t)
    u2 = u * u
    s = jnp.float32(1.0 / 13.0)
    for c in (1.0 / 11.0, 1.0 / 9.0, 1.0 / 7.0, 1.0 / 5.0, 1.0 / 3.0, 1.0):
        s = s * u2 + jnp.float32(c)
    return 2.0 * u * s


_mesh = plsc.VectorSubcoreMesh(
    core_axis_name="c", subcore_axis_name="s", num_cores=NC, num_subcores=NS)


@functools.partial(
    pl.kernel,
    out_type=jax.ShapeDtypeStruct((NW, L), jnp.float32),
    mesh=_mesh,
    scratch_types=[
        pltpu.VMEM((N,), jnp.float32),
        pltpu.VMEM((N,), jnp.float32),
        pltpu.VMEM((N + L,), jnp.float32),
        pltpu.VMEM((L,), jnp.float32),
    ],
)
def _sc_kernel(x_hbm, z_hbm, out_hbm, x_v, z_v, g_v, p_v):
    wid = lax.axis_index("s") * NC + lax.axis_index("c")
    pltpu.sync_copy(x_hbm, x_v)
    pltpu.sync_copy(z_hbm, z_v)

    def g_body(k, carry):
        x = x_v[pl.ds(k * L, L)]
        z = z_v[pl.ds(k * L, L)]
        pred = 1.0 / (1.0 + jnp.exp(-x))
        g_v[pl.ds(k * L, L)] = jnp.abs(pred - z)
        return carry

    lax.fori_loop(0, N // L, g_body, 0)
    g_v[pl.ds(N, L)] = g_v[pl.ds(0, L)]

    base = NTC + wid * CHUNK
    gi = [g_v[pl.ds(base + b * L, L)] for b in range(IV)]

    def j_body(t, accs):
        accs = list(accs)
        for r in range(UR):
            gw = g_v[pl.ds(t * UR + r, L)]
            for b in range(IV):
                m = jnp.abs(gw - gi[b]) <= DELTA
                accs[b] = accs[b] + jnp.where(m, 1.0, 0.0)
        return tuple(accs)

    zero = jnp.zeros((L,), jnp.float32)
    accs = lax.fori_loop(0, N // UR, j_body, tuple(zero for _ in range(IV)))

    psum = jnp.zeros((L,), jnp.float32)
    for b in range(IV):
        gd = accs[b] / DELTA
        beta = N / (gd + EPS)
        x = x_v[pl.ds(base + b * L, L)]
        z = z_v[pl.ds(base + b * L, L)]
        loss = jnp.maximum(x, 0.0) - x * z + _log1p_exp_neg(jnp.abs(x))
        psum = psum + beta * loss
    p_v[...] = psum
    pltpu.sync_copy(p_v, out_hbm.at[wid])


def _tc_body(xcb_ref, zcb_ref, xr_ref, zr_ref, out_ref):
    xc = xcb_ref[...].reshape(BI, 1)
    zc = zcb_ref[...].reshape(BI, 1)
    xr = xr_ref[...]
    zr = zr_ref[...]
    gc = jnp.abs(jax.nn.sigmoid(xc) - zc)
    gr = jnp.abs(jax.nn.sigmoid(xr) - zr)
    close = (jnp.abs(gc - gr) <= DELTA).astype(jnp.float32)
    cnt = jnp.sum(close, axis=1, keepdims=True)
    gd = cnt / DELTA
    beta = N / (gd + EPS)
    loss = jnp.maximum(xc, 0.0) - xc * zc + jnp.log1p(jnp.exp(-jnp.abs(xc)))

    @pl.when(pl.program_id(0) == 0)
    def _():
        out_ref[...] = jnp.zeros((1, 1, 1), jnp.float32)

    out_ref[...] += jnp.sum(beta * loss).reshape(1, 1, 1)


def kernel(logits, targets):
    sc_partials = _sc_kernel(logits, targets)

    xr = logits.reshape(1, N)
    zr = targets.reshape(1, N)
    tc_partials = pl.pallas_call(
        _tc_body,
        grid=(NTC // BI,),
        in_specs=[
            pl.BlockSpec((1, BI), lambda i: (0, i)),
            pl.BlockSpec((1, BI), lambda i: (0, i)),
            pl.BlockSpec((1, N), lambda i: (0, 0)),
            pl.BlockSpec((1, N), lambda i: (0, 0)),
        ],
        out_specs=pl.BlockSpec((1, 1, 1), lambda i: (0, 0, 0)),
        out_shape=jax.ShapeDtypeStruct((1, 1, 1), jnp.float32),
    )(xr, zr, xr, zr)
    return (jnp.sum(sc_partials) + tc_partials[0, 0, 0]) / N
